# baseline (speedup 1.0000x reference)
import functools

import jax
import jax.numpy as jnp
from jax import lax
from jax.experimental import pallas as pl
from jax.experimental.pallas import tpu as pltpu
from jax.experimental.pallas import tpu_sc as plsc

_EMB = 64
_SEQ = 50
_BB = 128
_NB = 5


@functools.lru_cache(maxsize=None)
def _make_gather():
    info = plsc.get_sparse_core_info()
    nc, ns = info.num_cores, info.num_subcores
    nw = nc * ns

    mesh = plsc.VectorSubcoreMesh(core_axis_name="c", subcore_axis_name="s")

    @functools.partial(
        pl.kernel,
        mesh=mesh,
        compiler_params=pltpu.CompilerParams(
            use_tc_tiling_on_sc=False, needs_layout_passes=False
        ),
        out_type=jax.ShapeDtypeStruct((_SEQ, _EMB // 8, nw, 8 * _BB), jnp.float32),
        scratch_types=(
            [pltpu.VMEM((_SEQ, _BB), jnp.int32)]
            + [pltpu.VMEM((_BB, _EMB), jnp.float32) for _ in range(_NB)]
            + [pltpu.VMEM((_EMB * _BB,), jnp.float32) for _ in range(_NB)]
            + [pltpu.SemaphoreType.DMA for _ in range(2 * _NB)]
        ),
    )
    def gather(tbl, idx_hbm, out5, idx_v, *bufs):
        rbs = bufs[:_NB]
        tbs = bufs[_NB:2 * _NB]
        gss = bufs[2 * _NB:3 * _NB]
        sss = bufs[3 * _NB:4 * _NB]
        w = lax.axis_index("s") * nc + lax.axis_index("c")
        pltpu.sync_copy(idx_hbm.at[w], idx_v)
        iot = lax.iota(jnp.int32, 16)
        rows = [iot + 16 * k for k in range(8)]

        def fire(s, rb, gs):
            pltpu.async_copy(tbl.at[idx_v.at[s]], rb, gs)

        def drain_gather(s, rb, gs):
            pltpu.make_async_copy(tbl.at[idx_v.at[s]], rb, gs).wait()

        def transpose(rs, td):
            for e in range(0, _EMB, 2):
                cols = [jnp.full((16,), e + d, jnp.int32) for d in range(2)]
                vs = [
                    plsc.load_gather(rs, [rows[k], cols[d]])
                    for d in range(2)
                    for k in range(8)
                ]
                for i, v in enumerate(vs):
                    d, k = divmod(i, 8)
                    td[pl.ds((e + d) * _BB + 16 * k, 16)] = v

        def fire_stores(s, td, ss):
            for e8 in range(8):
                pltpu.async_copy(td.at[pl.ds(8 * _BB * e8, 8 * _BB)], out5.at[s, e8, w], ss)

        def drain_stores(s_prev, td, ss):
            for e8 in range(8):
                pltpu.make_async_copy(
                    td.at[pl.ds(8 * _BB * e8, 8 * _BB)], out5.at[s_prev, e8, w], ss
                ).wait()

        def slot(i, s, j):
            drain_gather(s, rbs[j], gss[j])
            @pl.when(i > 0)
            def _():
                drain_stores(s - _NB, tbs[j], sss[j])
            transpose(rbs[j], tbs[j])
            @pl.when(s + _NB < _SEQ)
            def _():
                fire(s + _NB, rbs[j], gss[j])
            fire_stores(s, tbs[j], sss[j])

        for j in range(_NB):
            fire(j, rbs[j], gss[j])

        def body(i, carry):
            for j in range(_NB):
                slot(i, _NB * i + j, j)
            return carry

        lax.fori_loop(0, _SEQ // _NB, body, 0)
        for j in range(_NB):
            drain_stores(_SEQ - _NB + j, tbs[j], sss[j])

    return gather


def kernel(input, lang, table_0, table_1):
    n_batch, seq = input.shape
    nw = 32
    assert seq == _SEQ and n_batch == nw * _BB
    gather = _make_gather()
    idx_t = input.T.reshape(_SEQ, nw, _BB).transpose(1, 0, 2)
    sel = lang.reshape(-1)[0]
    out5 = lax.cond(
        sel == 0,
        lambda: gather(table_0, idx_t),
        lambda: gather(table_1, idx_t),
    )
    out6 = out5.reshape(_SEQ, _EMB // 8, nw, 8, _BB)
    return out6.transpose(2, 4, 0, 1, 3).reshape(n_batch, _SEQ, _EMB)


# speedup vs baseline: 1.4021x; 1.4021x over previous
import functools

import jax
import jax.numpy as jnp
from jax import lax
from jax.experimental import pallas as pl
from jax.experimental.pallas import tpu as pltpu
from jax.experimental.pallas import tpu_sc as plsc

_EMB = 64
_SEQ = 50
_BB = 128
_NB = 5
_PIT = 129


@functools.lru_cache(maxsize=None)
def _make_gather():
    info = plsc.get_sparse_core_info()
    nc, ns = info.num_cores, info.num_subcores
    nw = nc * ns

    mesh = plsc.VectorSubcoreMesh(core_axis_name="c", subcore_axis_name="s")

    @functools.partial(
        pl.kernel,
        mesh=mesh,
        compiler_params=pltpu.CompilerParams(
            use_tc_tiling_on_sc=False, needs_layout_passes=False
        ),
        out_type=jax.ShapeDtypeStruct((_SEQ, _EMB // 8, nw, 8, _BB), jnp.float32),
        scratch_types=(
            [pltpu.VMEM((_SEQ, _BB), jnp.int32)]
            + [pltpu.VMEM((_BB, _EMB), jnp.float32) for _ in range(_NB)]
            + [pltpu.VMEM((_EMB, _PIT), jnp.float32) for _ in range(_NB)]
            + [pltpu.SemaphoreType.DMA for _ in range(2 * _NB)]
        ),
    )
    def gather(tbl, idx_hbm, out5, idx_v, *bufs):
        rbs = bufs[:_NB]
        tbs = bufs[_NB:2 * _NB]
        gss = bufs[2 * _NB:3 * _NB]
        sss = bufs[3 * _NB:4 * _NB]
        w = lax.axis_index("s") * nc + lax.axis_index("c")
        pltpu.sync_copy(idx_hbm.at[w], idx_v)
        iot = lax.iota(jnp.int32, 16)
        rows = [iot + 16 * k for k in range(4)]

        def fire(s, rb, gs):
            pltpu.async_copy(tbl.at[idx_v.at[s]], rb, gs)

        def drain_gather(s, rb, gs):
            pltpu.make_async_copy(tbl.at[idx_v.at[s]], rb, gs).wait()

        def transpose(rs, td):
            for j0 in range(0, _BB, 4):
                vs = [
                    rs[j0 + dj, pl.ds(16 * k, 16)]
                    for dj in range(4)
                    for k in range(4)
                ]
                for i, v in enumerate(vs):
                    dj, k = divmod(i, 4)
                    colv = jnp.full((16,), j0 + dj, jnp.int32)
                    plsc.store_scatter(td, [rows[k], colv], v)

        def fire_stores(s, td, ss):
            for e8 in range(8):
                pltpu.async_copy(
                    td.at[pl.ds(8 * e8, 8), pl.ds(0, _BB)], out5.at[s, e8, w], ss
                )

        def drain_stores(s_prev, td, ss):
            for e8 in range(8):
                pltpu.make_async_copy(
                    td.at[pl.ds(8 * e8, 8), pl.ds(0, _BB)], out5.at[s_prev, e8, w], ss
                ).wait()

        def slot(i, s, j):
            drain_gather(s, rbs[j], gss[j])
            @pl.when(i > 0)
            def _():
                drain_stores(s - _NB, tbs[j], sss[j])
            transpose(rbs[j], tbs[j])
            @pl.when(s + _NB < _SEQ)
            def _():
                fire(s + _NB, rbs[j], gss[j])
            fire_stores(s, tbs[j], sss[j])

        for j in range(_NB):
            fire(j, rbs[j], gss[j])

        def body(i, carry):
            for j in range(_NB):
                slot(i, _NB * i + j, j)
            return carry

        lax.fori_loop(0, _SEQ // _NB, body, 0)
        for j in range(_NB):
            drain_stores(_SEQ - _NB + j, tbs[j], sss[j])

    return gather


def kernel(input, lang, table_0, table_1):
    n_batch, seq = input.shape
    nw = 32
    assert seq == _SEQ and n_batch == nw * _BB
    gather = _make_gather()
    idx_t = input.T.reshape(_SEQ, nw, _BB).transpose(1, 0, 2)
    sel = lang.reshape(-1)[0]
    out5 = lax.cond(
        sel == 0,
        lambda: gather(table_0, idx_t),
        lambda: gather(table_1, idx_t),
    )
    return out5.transpose(2, 4, 0, 1, 3).reshape(n_batch, _SEQ, _EMB)
